# Initial kernel scaffold; baseline (speedup 1.0000x reference)
import jax
import jax.numpy as jnp
from jax.experimental import pallas as pl


def kernel(node_embed, context_embed, walks, negatives):
    raise NotImplementedError("write your pallas kernel here")


# speedup vs baseline: 6.3376x; 6.3376x over previous
import functools

import jax
import jax.numpy as jnp
from jax import lax
from jax.experimental import pallas as pl
from jax.experimental.pallas import tpu as pltpu
from jax.experimental.pallas import tpu_sc as plsc

N_LANE = 16
D = 128
ND = D // N_LANE
WINDOW = 3
SLOT = 8


def _sc_scores(node_embed, context_embed, walks, neg_flat):
    B, L = walks.shape
    KN = neg_flat.shape[1]
    K = KN // L
    LP = L + (L % 2)
    SW = LP * SLOT
    info = plsc.get_sparse_core_info()
    NW = info.num_cores * info.num_subcores
    WPW = B // NW

    mesh = plsc.VectorSubcoreMesh(core_axis_name="c", subcore_axis_name="s")

    @functools.partial(
        pl.kernel,
        out_type=jax.ShapeDtypeStruct((B, SW), jnp.float32),
        mesh=mesh,
        scratch_types=[
            pltpu.VMEM((WPW, L), jnp.int32),
            pltpu.VMEM((WPW, KN), jnp.int32),
            pltpu.VMEM((L, D), jnp.float32),
            pltpu.VMEM((L, D), jnp.float32),
            pltpu.VMEM((KN, D), jnp.float32),
            pltpu.VMEM((SW,), jnp.float32),
            pltpu.SemaphoreType.DMA,
            pltpu.SemaphoreType.DMA,
            pltpu.SemaphoreType.DMA,
        ],
    )
    def k(ne_hbm, ce_hbm, walks_hbm, neg_hbm, out_hbm,
          widx, nidx, eu, cv, nv, sb, s0, s1, s2):
        wid = lax.axis_index("s") * info.num_cores + lax.axis_index("c")
        base = wid * WPW
        pltpu.sync_copy(walks_hbm.at[pl.ds(base, WPW)], widx)
        pltpu.sync_copy(neg_hbm.at[pl.ds(base, WPW)], nidx)

        lane = lax.iota(jnp.int32, N_LANE)
        onehot = [lane == i for i in range(N_LANE)]

        def walk_body(w, carry):
            c0 = pltpu.async_copy(ne_hbm.at[widx.at[w]], eu, s0)
            c1 = pltpu.async_copy(ce_hbm.at[widx.at[w]], cv, s1)
            c2 = pltpu.async_copy(ce_hbm.at[nidx.at[w]], nv, s2)
            c0.wait()
            c1.wait()
            c2.wait()

            def dotv(vref, row, u):
                acc = u[0] * vref[row, pl.ds(0, N_LANE)]
                for j in range(1, ND):
                    acc = acc + u[j] * vref[row, pl.ds(j * N_LANE, N_LANE)]
                for sh in (8, 4, 2, 1):
                    acc = acc + acc.at[lane ^ sh].get(
                        mode="promise_in_bounds", unique_indices=True)
                return acc

            def l2_body(l2, carry2):
                sv = jnp.zeros((N_LANE,), jnp.float32)
                for dl in range(2):
                    l = l2 * 2 + dl
                    lc = jnp.minimum(l, L - 1)
                    u = [eu[lc, pl.ds(j * N_LANE, N_LANE)] for j in range(ND)]
                    for off in range(1, WINDOW + 1):
                        r = jnp.minimum(lc + off, L - 1)
                        tot = dotv(cv, r, u)
                        sv = jnp.where(onehot[dl * SLOT + off - 1], tot, sv)
                    for kk in range(K):
                        tot = dotv(nv, lc * K + kk, u)
                        sv = jnp.where(onehot[dl * SLOT + WINDOW + kk], tot, sv)
                sb[pl.ds(l2 * N_LANE, N_LANE)] = sv
                return carry2

            lax.fori_loop(0, LP // 2, l2_body, 0)
            pltpu.sync_copy(sb, out_hbm.at[base + w])
            return carry

        lax.fori_loop(0, WPW, walk_body, 0)

    return k(node_embed, context_embed, walks, neg_flat)


def _tc_loss(scores, L):
    B, SW = scores.shape

    def body(s_ref, o_ref):
        s = s_ref[...]
        col = lax.broadcasted_iota(jnp.int32, s.shape, 1)
        l = col // SLOT
        slot = col % SLOT
        is_pos = slot < WINDOW
        valid = (is_pos & ((l + slot + 1) < L)) | (~is_pos & (l < L))
        t = jnp.where(is_pos, s, -s)
        ls = jnp.minimum(t, 0.0) - jnp.log1p(jnp.exp(-jnp.abs(t)))
        contrib = jnp.where(valid, -ls, 0.0)
        o_ref[0, 0] = jnp.sum(contrib) / B

    return pl.pallas_call(
        body,
        out_shape=jax.ShapeDtypeStruct((1, 1), jnp.float32),
        out_specs=pl.BlockSpec(memory_space=pltpu.SMEM),
    )(scores)


def kernel(node_embed, context_embed, walks, negatives):
    B, L = walks.shape
    K = negatives.shape[-1]
    w = jnp.maximum(walks.astype(jnp.int32), 0)
    n = negatives.astype(jnp.int32).reshape(B, L * K)
    scores = _sc_scores(node_embed, context_embed, w, n)
    loss = _tc_loss(scores, L)
    return loss[0, 0]


# speedup vs baseline: 11.4171x; 1.8015x over previous
import functools

import jax
import jax.numpy as jnp
from jax import lax
from jax.experimental import pallas as pl
from jax.experimental.pallas import tpu as pltpu
from jax.experimental.pallas import tpu_sc as plsc

N_LANE = 16
D = 128
ND = D // N_LANE
WINDOW = 3
SLOT = 8


def _sc_scores(node_embed, context_embed, walks, neg_flat):
    B, L = walks.shape
    KN = neg_flat.shape[1]
    K = KN // L
    LP = L + (L % 2)
    SW = LP * SLOT
    info = plsc.get_sparse_core_info()
    NW = info.num_cores * info.num_subcores
    WPW = B // NW

    mesh = plsc.VectorSubcoreMesh(core_axis_name="c", subcore_axis_name="s")

    @functools.partial(
        pl.kernel,
        out_type=jax.ShapeDtypeStruct((B, SW), jnp.float32),
        mesh=mesh,
        scratch_types=[
            pltpu.VMEM((WPW, L), jnp.int32),
            pltpu.VMEM((WPW, KN), jnp.int32),
            pltpu.VMEM((2, L, D), jnp.float32),
            pltpu.VMEM((2, L, D), jnp.float32),
            pltpu.VMEM((2, KN, D), jnp.float32),
            pltpu.VMEM((2, SW), jnp.float32),
            pltpu.SemaphoreType.DMA,
            pltpu.SemaphoreType.DMA,
            pltpu.SemaphoreType.DMA,
            pltpu.SemaphoreType.DMA,
            pltpu.SemaphoreType.DMA,
            pltpu.SemaphoreType.DMA,
            pltpu.SemaphoreType.DMA,
            pltpu.SemaphoreType.DMA,
        ],
    )
    def k(ne_hbm, ce_hbm, walks_hbm, neg_hbm, out_hbm,
          widx, nidx, eu, cv, nv, sb, g0a, g0b, g0c, g1a, g1b, g1c, o0, o1):
        wid = lax.axis_index("s") * info.num_cores + lax.axis_index("c")
        base = wid * WPW
        gsem = ((g0a, g0b, g0c), (g1a, g1b, g1c))
        osem = (o0, o1)
        pltpu.sync_copy(walks_hbm.at[pl.ds(base, WPW)], widx)
        pltpu.sync_copy(neg_hbm.at[pl.ds(base, WPW)], nidx)

        lane = lax.iota(jnp.int32, N_LANE)
        onehot = [lane == i for i in range(N_LANE)]

        def start_gathers(w, slot):
            return (
                pltpu.async_copy(ne_hbm.at[widx.at[w]], eu.at[slot],
                                 gsem[slot][0]),
                pltpu.async_copy(ce_hbm.at[widx.at[w]], cv.at[slot],
                                 gsem[slot][1]),
                pltpu.async_copy(ce_hbm.at[nidx.at[w]], nv.at[slot],
                                 gsem[slot][2]),
            )

        def compute_scores(slot):
            def dotv(vref, row, u):
                acc = u[0] * vref[slot, row, pl.ds(0, N_LANE)]
                for j in range(1, ND):
                    acc = acc + u[j] * vref[slot, row, pl.ds(j * N_LANE, N_LANE)]
                for sh in (8, 4, 2, 1):
                    acc = acc + acc.at[lane ^ sh].get(
                        mode="promise_in_bounds", unique_indices=True)
                return acc

            def l2_body(l2, carry2):
                sv = jnp.zeros((N_LANE,), jnp.float32)
                for dl in range(2):
                    l = l2 * 2 + dl
                    lc = jnp.minimum(l, L - 1)
                    u = [eu[slot, lc, pl.ds(j * N_LANE, N_LANE)]
                         for j in range(ND)]
                    for off in range(1, WINDOW + 1):
                        r = jnp.minimum(lc + off, L - 1)
                        tot = dotv(cv, r, u)
                        sv = jnp.where(onehot[dl * SLOT + off - 1], tot, sv)
                    for kk in range(K):
                        tot = dotv(nv, lc * K + kk, u)
                        sv = jnp.where(onehot[dl * SLOT + WINDOW + kk], tot, sv)
                sb[slot, pl.ds(l2 * N_LANE, N_LANE)] = sv
                return carry2

            lax.fori_loop(0, LP // 2, l2_body, 0)

        for h in start_gathers(0, 0):
            h.wait()

        def body(i, carry):
            for dl in range(2):
                w = 2 * i + dl
                slot = dl
                hs = start_gathers(jnp.minimum(w + 1, WPW - 1), 1 - slot)
                @pl.when(i > 0)
                def _():
                    pltpu.make_async_copy(sb.at[slot],
                                          out_hbm.at[base + w - 2],
                                          osem[slot]).wait()
                compute_scores(slot)
                pltpu.async_copy(sb.at[slot], out_hbm.at[base + w], osem[slot])
                for h in hs:
                    h.wait()
            return carry

        lax.fori_loop(0, WPW // 2, body, 0)
        for slot in range(2):
            pltpu.make_async_copy(sb.at[slot],
                                  out_hbm.at[base + WPW - 2 + slot],
                                  osem[slot]).wait()

    return k(node_embed, context_embed, walks, neg_flat)


def _tc_loss(scores, L):
    B, SW = scores.shape

    def body(s_ref, o_ref):
        s = s_ref[...]
        col = lax.broadcasted_iota(jnp.int32, s.shape, 1)
        l = col // SLOT
        slot = col % SLOT
        is_pos = slot < WINDOW
        valid = (is_pos & ((l + slot + 1) < L)) | (~is_pos & (l < L))
        t = jnp.where(is_pos, s, -s)
        ls = jnp.minimum(t, 0.0) - jnp.log1p(jnp.exp(-jnp.abs(t)))
        contrib = jnp.where(valid, -ls, 0.0)
        o_ref[0, 0] = jnp.sum(contrib) / B

    return pl.pallas_call(
        body,
        out_shape=jax.ShapeDtypeStruct((1, 1), jnp.float32),
        out_specs=pl.BlockSpec(memory_space=pltpu.SMEM),
    )(scores)


def kernel(node_embed, context_embed, walks, negatives):
    B, L = walks.shape
    K = negatives.shape[-1]
    w = jnp.maximum(walks.astype(jnp.int32), 0)
    n = negatives.astype(jnp.int32).reshape(B, L * K)
    scores = _sc_scores(node_embed, context_embed, w, n)
    loss = _tc_loss(scores, L)
    return loss[0, 0]
